# baseline (speedup 1.0000x reference)
import functools

import jax
import jax.numpy as jnp
from jax import lax
from jax.experimental import pallas as pl
from jax.experimental.pallas import tpu as pltpu
from jax.experimental.pallas import tpu_sc as plsc

B, N, D = 4096, 200, 128
NC, NS, L = 2, 16, 16
NW = NC * NS
HALVES = 2
HB = B // HALVES
BPW = HB // NW


def _sc_gather_body(h, table, idx1_hbm, idx2_hbm, ll_hbm,
                    idx_v1, idx_v2, rows1, rows2, sem1, sem2, sem3, sem4):
    wid = lax.axis_index("s") * NC + lax.axis_index("c")
    lbase = wid * BPW
    base = h * HB + lbase
    ci1 = pltpu.async_copy(idx1_hbm.at[pl.ds(base, BPW)], idx_v1, sem1)
    ci2 = pltpu.async_copy(idx2_hbm.at[pl.ds(base, BPW)], idx_v2, sem2)
    ci1.wait()
    ci2.wait()
    lane = lax.iota(jnp.int32, L) * N
    for i in range(BPW // L):
        off = lane + (base + i * L) * N
        sl = pl.ds(i * L, L)
        idx_v1[sl] = idx_v1[sl] + off
        idx_v2[sl] = idx_v2[sl] + off
    c1 = pltpu.async_copy(table.at[idx_v1], rows1, sem1)
    c2 = pltpu.async_copy(table.at[idx_v2], rows2, sem2)
    c1.wait()
    o1 = pltpu.async_copy(rows1, ll_hbm.at[pl.ds(lbase, BPW), pl.ds(0, D)],
                          sem3)
    c2.wait()
    o2 = pltpu.async_copy(rows2, ll_hbm.at[pl.ds(lbase, BPW), pl.ds(D, D)],
                          sem4)
    o1.wait()
    o2.wait()


@functools.cache
def _sc_gather(h):
    return pl.kernel(
        functools.partial(_sc_gather_body, h),
        out_type=jax.ShapeDtypeStruct((HB, 2 * D), jnp.float32),
        mesh=plsc.VectorSubcoreMesh(
            core_axis_name="c", subcore_axis_name="s",
            num_cores=NC, num_subcores=NS,
        ),
        scratch_types=[
            pltpu.VMEM((BPW,), jnp.int32),
            pltpu.VMEM((BPW,), jnp.int32),
            pltpu.VMEM((BPW, D), jnp.float32),
            pltpu.VMEM((BPW, D), jnp.float32),
            pltpu.SemaphoreType.DMA,
            pltpu.SemaphoreType.DMA,
            pltpu.SemaphoreType.DMA,
            pltpu.SemaphoreType.DMA,
        ],
    )


def _tc_linear_first_body(ll_ref, r1_ref, r2_ref, wr_ref, wl_ref, out_ref):
    wf = jnp.dot(wr_ref[...], wl_ref[2 * D:, :],
                 preferred_element_type=jnp.float32)
    ratio = r1_ref[...] / r2_ref[...]
    acc = jnp.dot(ll_ref[...], wl_ref[:2 * D, :],
                  preferred_element_type=jnp.float32)
    out_ref[...] = acc + ratio * wf


def _tc_linear_second_body(prev_ref, ll_ref, r1_ref, r2_ref, wr_ref, wl_ref,
                           out_ref):
    del prev_ref
    _tc_linear_first_body(ll_ref, r1_ref, r2_ref, wr_ref, wl_ref, out_ref)


@functools.cache
def _tc_linear(h):
    common_in_specs = [
        pl.BlockSpec((HB, 2 * D), lambda i: (0, 0)),
        pl.BlockSpec((HB, 1), lambda i, h=h: (h, 0)),
        pl.BlockSpec((HB, 1), lambda i, h=h: (h, 0)),
        pl.BlockSpec((1, D), lambda i: (0, 0)),
        pl.BlockSpec((3 * D, D), lambda i: (0, 0)),
    ]
    if h == 0:
        body = _tc_linear_first_body
        in_specs = common_in_specs
        aliases = {}
    else:
        body = _tc_linear_second_body
        in_specs = [pl.BlockSpec((8, D), lambda i: (0, 0))] + common_in_specs
        aliases = {0: 0}
    return pl.pallas_call(
        body,
        grid=(1,),
        in_specs=in_specs,
        out_specs=pl.BlockSpec((HB, D), lambda i, h=h: (h, 0)),
        out_shape=jax.ShapeDtypeStruct((B, D), jnp.float32),
        input_output_aliases=aliases,
    )


def kernel(encoded_row, loadlock1_wafer_in, loadlock2_wafer_in, W_ratio,
           W_lin, loadlock1_wafer_recipe, loadlock2_wafer_recipe):
    table = encoded_row.reshape(B * N, D)
    ll0 = _sc_gather(0)(table, loadlock1_wafer_recipe, loadlock2_wafer_recipe)
    ll1 = _sc_gather(1)(table, loadlock1_wafer_recipe, loadlock2_wafer_recipe)
    out = _tc_linear(0)(ll0, loadlock1_wafer_in, loadlock2_wafer_in,
                        W_ratio, W_lin)
    return _tc_linear(1)(out, ll1, loadlock1_wafer_in, loadlock2_wafer_in,
                         W_ratio, W_lin)
